# baseline (speedup 1.0000x reference)
import functools

import jax
import jax.numpy as jnp
from jax import lax
from jax.experimental import pallas as pl
from jax.experimental.pallas import tpu as pltpu
from jax.experimental.pallas import tpu_sc as plsc

H = 4096
W = 4096
KS = 4
B = 16
HB = H // B
WB = W // B
N = 8192
BLK = B * B * KS
PAD_N = N + 64
NB = 2 * HB
OFF_LEN = 528

NT = 16
CHUNK1 = N // NT

NW = 32
RPW = N // NW
CH2 = 64
TCH = 16


def _ranks(rows_v, tmp_v, lanes, i):
    rv = rows_v[pl.ds(i * 16, 16)]
    srt, lids = plsc.sort_key_val(rv, lanes)
    tmp_v[...] = srt
    prev = plsc.load_gather(tmp_v, [jnp.maximum(lanes - 1, 0)])
    nxt = plsc.load_gather(tmp_v, [jnp.minimum(lanes + 1, 15)])
    segs = (lanes == 0) | (srt != prev)
    last = (lanes == 15) | (srt != nxt)
    rank = lanes - plsc.cummax(jnp.where(segs, lanes, 0))
    return srt, lids, rank, last


def _sc_sort_body(rows_hbm, cols_hbm, off_hbm, cs_hbm, pos_hbm,
                  rows_v, cols_v, hist_v, base_v, pos_v, off_v,
                  allh_v, tmp_v, hists_sh, sem):
    t = lax.axis_index("s")
    base0 = t * CHUNK1
    pltpu.sync_copy(rows_hbm.at[pl.ds(base0, CHUNK1)], rows_v)
    pltpu.sync_copy(cols_hbm.at[pl.ds(base0, CHUNK1)], cols_v)

    lanes = lax.iota(jnp.int32, 16)
    zero16 = jnp.zeros((16,), jnp.int32)
    for k in range(CHUNK1 // 16):
        cv = cols_v[pl.ds(k * 16, 16)]
        rows_v[pl.ds(k * 16, 16)] = (
            rows_v[pl.ds(k * 16, 16)] * 2 + (cv & 1))
        cols_v[pl.ds(k * 16, 16)] = (cv >> 1) * 128
    for k in range(NB // 16):
        hist_v[pl.ds(k * 16, 16)] = zero16

    def hist_body(i, c):
        srt, _, rank, last = _ranks(rows_v, tmp_v, lanes, i)
        plsc.addupdate_scatter(hist_v, [srt], rank + 1, mask=last)
        return c
    lax.fori_loop(0, CHUNK1 // 16, hist_body, 0)

    pltpu.sync_copy(hist_v, hists_sh.at[t])
    plsc.subcore_barrier()
    pltpu.sync_copy(hists_sh, allh_v)

    run = jnp.int32(0)
    for rg in range(NB // 16):
        tot = zero16
        before = zero16
        for tp in range(NT):
            row = allh_v[tp, pl.ds(rg * 16, 16)]
            tot = tot + row
            before = before + row * (jnp.int32(tp) < t).astype(jnp.int32)
        excl = plsc.cumsum(tot) - tot
        off_v[pl.ds(rg * 16, 16)] = run + excl
        base_v[pl.ds(rg * 16, 16)] = run + excl + before
        run = run + jnp.sum(tot)
    for k in range(NB, OFF_LEN, 16):
        off_v[pl.ds(k, 16)] = jnp.full((16,), N, jnp.int32)

    def pos_body(i, c):
        srt, lids, rank, last = _ranks(rows_v, tmp_v, lanes, i)
        p = plsc.load_gather(base_v, [srt]) + rank
        plsc.addupdate_scatter(base_v, [srt], rank + 1, mask=last)
        plsc.store_scatter(pos_v, [i * 16 + lids], p)
        return c
    lax.fori_loop(0, CHUNK1 // 16, pos_body, 0)

    @pl.when(t == 0)
    def _():
        pltpu.sync_copy(off_v, off_hbm)

    pltpu.async_copy(cols_v, cs_hbm.at[pos_v], sem).wait()
    pltpu.sync_copy(pos_v, pos_hbm.at[pl.ds(base0, CHUNK1)])


@functools.cache
def _sc_sort():
  return pl.kernel(
    _sc_sort_body,
    out_type=(jax.ShapeDtypeStruct((OFF_LEN,), jnp.int32),
              jax.ShapeDtypeStruct((N,), jnp.int32),
              jax.ShapeDtypeStruct((N,), jnp.int32)),
    mesh=plsc.VectorSubcoreMesh(core_axis_name="c", subcore_axis_name="s",
                                num_cores=1, num_subcores=NT),
    scratch_types=[
        pltpu.VMEM((CHUNK1,), jnp.int32),
        pltpu.VMEM((CHUNK1,), jnp.int32),
        pltpu.VMEM((NB,), jnp.int32),
        pltpu.VMEM((NB,), jnp.int32),
        pltpu.VMEM((CHUNK1,), jnp.int32),
        pltpu.VMEM((OFF_LEN,), jnp.int32),
        pltpu.VMEM((NT, NB), jnp.int32),
        pltpu.VMEM((16,), jnp.int32),
        pltpu.VMEM_SHARED((NT, NB), jnp.int32),
        pltpu.SemaphoreType.DMA,
    ],
    compiler_params=pltpu.CompilerParams(needs_layout_passes=False),
  )


def _sc_permute_body(pos_hbm, bv_hbm, bs_hbm, posc_v, buf_v, sem):
    c = lax.axis_index("c")
    s = lax.axis_index("s")
    wid = s * 2 + c
    for k in range(RPW // CH2):
        base = wid * RPW + k * CH2
        pltpu.sync_copy(pos_hbm.at[pl.ds(base, CH2)], posc_v)
        pltpu.sync_copy(bv_hbm.at[pl.ds(base, CH2)], buf_v)
        pltpu.async_copy(buf_v, bs_hbm.at[posc_v], sem).wait()


@functools.cache
def _sc_permute():
  return pl.kernel(
    _sc_permute_body,
    out_type=jax.ShapeDtypeStruct((PAD_N, BLK), jnp.float32),
    mesh=plsc.VectorSubcoreMesh(core_axis_name="c", subcore_axis_name="s",
                                num_cores=2, num_subcores=NT),
    scratch_types=[
        pltpu.VMEM((CH2,), jnp.int32),
        pltpu.VMEM((CH2, BLK), jnp.float32),
        pltpu.SemaphoreType.DMA,
    ],
  )


def _tc_body(off_smem, cs_smem, bs_any, out_any, chunk_v, acc_v,
             sem_in, sem_out):
    i = pl.program_id(0)
    n0 = off_smem[2 * i]
    nmid = off_smem[2 * i + 1]
    cnt = off_smem[2 * i + 2] - n0
    nch = (cnt + TCH - 1) // TCH
    sel = i % 3

    @pl.when((i == 0) & (nch > 0))
    def _():
        pltpu.make_async_copy(bs_any.at[pl.ds(n0, TCH)], chunk_v.at[2],
                              sem_in.at[2]).start()

    @pl.when(i >= 3)
    def _():
        pltpu.make_async_copy(acc_v.at[sel],
                              out_any.at[pl.ds((i - 3) * B, B)],
                              sem_out.at[sel]).wait()

    acc_v[sel] = jnp.zeros((B, W * KS), jnp.float32)
    zhalf = jnp.zeros((B, B * KS), jnp.float32)

    def chunk_body(ci, carry):
        csel = jnp.where(ci == 0, 2, (ci - 1) % 2)
        base = n0 + ci * TCH
        pltpu.make_async_copy(bs_any.at[pl.ds(base, TCH)], chunk_v.at[csel],
                              sem_in.at[csel]).wait()

        @pl.when(ci + 1 < nch)
        def _():
            pltpu.make_async_copy(bs_any.at[pl.ds(base + TCH, TCH)],
                                  chunk_v.at[ci % 2],
                                  sem_in.at[ci % 2]).start()

        m = jnp.minimum(cnt - ci * TCH, TCH)

        def q_body(q4, carry2):
            for k in range(4):
                qq = q4 * 4 + k
                g = base + qq
                lo = pl.multiple_of(cs_smem[jnp.minimum(g, N - 1)], 128)
                blk = chunk_v[csel, jnp.minimum(qq, TCH - 1)]
                even = g < nmid
                blk128 = jnp.where(
                    even,
                    jnp.concatenate([blk, zhalf], axis=1),
                    jnp.concatenate([zhalf, blk], axis=1))
                blk128 = jnp.where(qq < m, blk128, 0.0)
                acc_v[sel, :, pl.ds(lo, 128)] = (
                    acc_v[sel, :, pl.ds(lo, 128)] + blk128)
            return carry2
        lax.fori_loop(0, (m + 3) // 4, q_body, 0)
        return carry
    lax.fori_loop(0, nch, chunk_body, 0)

    @pl.when(i + 1 < HB)
    def _():
        n0n = off_smem[2 * i + 2]

        @pl.when(off_smem[2 * i + 4] > n0n)
        def _():
            pltpu.make_async_copy(bs_any.at[pl.ds(n0n, TCH)], chunk_v.at[2],
                                  sem_in.at[2]).start()

    pltpu.make_async_copy(acc_v.at[sel], out_any.at[pl.ds(i * B, B)],
                          sem_out.at[sel]).start()

    @pl.when(i == HB - 1)
    def _():
        for d in range(3):
            pltpu.make_async_copy(acc_v.at[(i - d) % 3],
                                  out_any.at[pl.ds((i - d) * B, B)],
                                  sem_out.at[(i - d) % 3]).wait()


def _tc_accum(off, cs, bs3):
    grid_spec = pltpu.PrefetchScalarGridSpec(
        num_scalar_prefetch=2,
        grid=(HB,),
        in_specs=[pl.BlockSpec(memory_space=pltpu.HBM)],
        out_specs=pl.BlockSpec(memory_space=pltpu.HBM),
        scratch_shapes=[
            pltpu.VMEM((3, TCH, B, B * KS), jnp.float32),
            pltpu.VMEM((3, B, W * KS), jnp.float32),
            pltpu.SemaphoreType.DMA((3,)),
            pltpu.SemaphoreType.DMA((3,)),
        ],
    )
    return pl.pallas_call(
        _tc_body,
        grid_spec=grid_spec,
        out_shape=jax.ShapeDtypeStruct((H, W * KS), jnp.float32),
        compiler_params=pltpu.CompilerParams(
            dimension_semantics=("arbitrary",)),
    )(off, cs, bs3)


def kernel(block_indices, block_values):
    rows = block_indices[:, 0]
    cols = block_indices[:, 1]
    bv2 = block_values.reshape(N, BLK)
    off, cs, pos = _sc_sort()(rows, cols)
    bs = _sc_permute()(pos, bv2)
    out2 = _tc_accum(off, cs, bs.reshape(PAD_N, B, B * KS))
    return out2.reshape(H, W, KS)


# speedup vs baseline: 1.0910x; 1.0910x over previous
import functools

import jax
import jax.numpy as jnp
from jax import lax
from jax.experimental import pallas as pl
from jax.experimental.pallas import tpu as pltpu
from jax.experimental.pallas import tpu_sc as plsc

H = 4096
W = 4096
KS = 4
B = 16
HB = H // B
WB = W // B
N = 8192
BLK = B * B * KS
PAD_N = N + 64
NB = 2 * HB
OFF_LEN = 528

NT = 16
CHUNK1 = N // NT

NW = 32
RPW = N // NW
CH2 = 64
TCH = 32


def _ranks(rows_v, tmp_v, lanes, i):
    rv = rows_v[pl.ds(i * 16, 16)]
    srt, lids = plsc.sort_key_val(rv, lanes)
    tmp_v[...] = srt
    prev = plsc.load_gather(tmp_v, [jnp.maximum(lanes - 1, 0)])
    nxt = plsc.load_gather(tmp_v, [jnp.minimum(lanes + 1, 15)])
    segs = (lanes == 0) | (srt != prev)
    last = (lanes == 15) | (srt != nxt)
    rank = lanes - plsc.cummax(jnp.where(segs, lanes, 0))
    return srt, lids, rank, last


def _sc_sort_body(rows_hbm, cols_hbm, off_hbm, cs_hbm, pos_hbm,
                  rows_v, cols_v, hist_v, base_v, pos_v, off_v,
                  allh_v, tmp_v, hists_sh, sem):
    t = lax.axis_index("s")
    base0 = t * CHUNK1
    pltpu.sync_copy(rows_hbm.at[pl.ds(base0, CHUNK1)], rows_v)
    pltpu.sync_copy(cols_hbm.at[pl.ds(base0, CHUNK1)], cols_v)

    lanes = lax.iota(jnp.int32, 16)
    zero16 = jnp.zeros((16,), jnp.int32)
    for k in range(CHUNK1 // 16):
        cv = cols_v[pl.ds(k * 16, 16)]
        rows_v[pl.ds(k * 16, 16)] = (
            rows_v[pl.ds(k * 16, 16)] * 2 + (cv & 1))
        cols_v[pl.ds(k * 16, 16)] = (cv >> 1) * 128
    for k in range(NB // 16):
        hist_v[pl.ds(k * 16, 16)] = zero16

    def hist_body(i, c):
        srt, _, rank, last = _ranks(rows_v, tmp_v, lanes, i)
        plsc.addupdate_scatter(hist_v, [srt], rank + 1, mask=last)
        return c
    lax.fori_loop(0, CHUNK1 // 16, hist_body, 0)

    pltpu.sync_copy(hist_v, hists_sh.at[t])
    plsc.subcore_barrier()
    pltpu.sync_copy(hists_sh, allh_v)

    run = jnp.int32(0)
    for rg in range(NB // 16):
        tot = zero16
        before = zero16
        for tp in range(NT):
            row = allh_v[tp, pl.ds(rg * 16, 16)]
            tot = tot + row
            before = before + row * (jnp.int32(tp) < t).astype(jnp.int32)
        excl = plsc.cumsum(tot) - tot
        off_v[pl.ds(rg * 16, 16)] = run + excl
        base_v[pl.ds(rg * 16, 16)] = run + excl + before
        run = run + jnp.sum(tot)
    for k in range(NB, OFF_LEN, 16):
        off_v[pl.ds(k, 16)] = jnp.full((16,), N, jnp.int32)

    def pos_body(i, c):
        srt, lids, rank, last = _ranks(rows_v, tmp_v, lanes, i)
        p = plsc.load_gather(base_v, [srt]) + rank
        plsc.addupdate_scatter(base_v, [srt], rank + 1, mask=last)
        plsc.store_scatter(pos_v, [i * 16 + lids], p)
        return c
    lax.fori_loop(0, CHUNK1 // 16, pos_body, 0)

    @pl.when(t == 0)
    def _():
        pltpu.sync_copy(off_v, off_hbm)

    pltpu.async_copy(cols_v, cs_hbm.at[pos_v], sem).wait()
    pltpu.sync_copy(pos_v, pos_hbm.at[pl.ds(base0, CHUNK1)])


@functools.cache
def _sc_sort():
  return pl.kernel(
    _sc_sort_body,
    out_type=(jax.ShapeDtypeStruct((OFF_LEN,), jnp.int32),
              jax.ShapeDtypeStruct((N,), jnp.int32),
              jax.ShapeDtypeStruct((N,), jnp.int32)),
    mesh=plsc.VectorSubcoreMesh(core_axis_name="c", subcore_axis_name="s",
                                num_cores=1, num_subcores=NT),
    scratch_types=[
        pltpu.VMEM((CHUNK1,), jnp.int32),
        pltpu.VMEM((CHUNK1,), jnp.int32),
        pltpu.VMEM((NB,), jnp.int32),
        pltpu.VMEM((NB,), jnp.int32),
        pltpu.VMEM((CHUNK1,), jnp.int32),
        pltpu.VMEM((OFF_LEN,), jnp.int32),
        pltpu.VMEM((NT, NB), jnp.int32),
        pltpu.VMEM((16,), jnp.int32),
        pltpu.VMEM_SHARED((NT, NB), jnp.int32),
        pltpu.SemaphoreType.DMA,
    ],
    compiler_params=pltpu.CompilerParams(needs_layout_passes=False),
  )


def _sc_permute_body(pos_hbm, bv_hbm, bs_hbm, posc_v, buf_v, sem):
    c = lax.axis_index("c")
    s = lax.axis_index("s")
    wid = s * 2 + c
    for k in range(RPW // CH2):
        base = wid * RPW + k * CH2
        pltpu.sync_copy(pos_hbm.at[pl.ds(base, CH2)], posc_v)
        pltpu.sync_copy(bv_hbm.at[pl.ds(base, CH2)], buf_v)
        pltpu.async_copy(buf_v, bs_hbm.at[posc_v], sem).wait()


@functools.cache
def _sc_permute():
  return pl.kernel(
    _sc_permute_body,
    out_type=jax.ShapeDtypeStruct((PAD_N, BLK), jnp.float32),
    mesh=plsc.VectorSubcoreMesh(core_axis_name="c", subcore_axis_name="s",
                                num_cores=2, num_subcores=NT),
    scratch_types=[
        pltpu.VMEM((CH2,), jnp.int32),
        pltpu.VMEM((CH2, BLK), jnp.float32),
        pltpu.SemaphoreType.DMA,
    ],
  )


def _tc_body(off_smem, cs_smem, bs_any, out_any, chunk_v, acc_v,
             sem_in, sem_out):
    i = pl.program_id(0)
    n0 = off_smem[2 * i]
    nmid = off_smem[2 * i + 1]
    cnt = off_smem[2 * i + 2] - n0
    nch = (cnt + TCH - 1) // TCH
    sel = i % 3

    @pl.when((i == 0) & (nch > 0))
    def _():
        pltpu.make_async_copy(bs_any.at[pl.ds(n0, TCH)], chunk_v.at[2],
                              sem_in.at[2]).start()

    @pl.when(i >= 3)
    def _():
        pltpu.make_async_copy(acc_v.at[sel],
                              out_any.at[pl.ds((i - 3) * B, B)],
                              sem_out.at[sel]).wait()

    acc_v[sel] = jnp.zeros((B, W * KS), jnp.float32)
    zhalf = jnp.zeros((B, B * KS), jnp.float32)

    def chunk_body(ci, carry):
        csel = jnp.where(ci == 0, 2, (ci - 1) % 2)
        base = n0 + ci * TCH
        pltpu.make_async_copy(bs_any.at[pl.ds(base, TCH)], chunk_v.at[csel],
                              sem_in.at[csel]).wait()

        @pl.when(ci + 1 < nch)
        def _():
            pltpu.make_async_copy(bs_any.at[pl.ds(base + TCH, TCH)],
                                  chunk_v.at[ci % 2],
                                  sem_in.at[ci % 2]).start()

        m = jnp.minimum(cnt - ci * TCH, TCH)

        def q_body(q4, carry2):
            for k in range(4):
                qq = q4 * 4 + k
                g = base + qq
                lo = pl.multiple_of(cs_smem[jnp.minimum(g, N - 1)], 128)
                blk = chunk_v[csel, jnp.minimum(qq, TCH - 1)]
                even = g < nmid
                blk128 = jnp.where(
                    even,
                    jnp.concatenate([blk, zhalf], axis=1),
                    jnp.concatenate([zhalf, blk], axis=1))
                blk128 = jnp.where(qq < m, blk128, 0.0)
                acc_v[sel, :, pl.ds(lo, 128)] = (
                    acc_v[sel, :, pl.ds(lo, 128)] + blk128)
            return carry2
        lax.fori_loop(0, (m + 3) // 4, q_body, 0)
        return carry
    lax.fori_loop(0, nch, chunk_body, 0)

    @pl.when(i + 1 < HB)
    def _():
        n0n = off_smem[2 * i + 2]

        @pl.when(off_smem[2 * i + 4] > n0n)
        def _():
            pltpu.make_async_copy(bs_any.at[pl.ds(n0n, TCH)], chunk_v.at[2],
                                  sem_in.at[2]).start()

    pltpu.make_async_copy(acc_v.at[sel], out_any.at[pl.ds(i * B, B)],
                          sem_out.at[sel]).start()

    @pl.when(i == HB - 1)
    def _():
        for d in range(3):
            pltpu.make_async_copy(acc_v.at[(i - d) % 3],
                                  out_any.at[pl.ds((i - d) * B, B)],
                                  sem_out.at[(i - d) % 3]).wait()


def _tc_accum(off, cs, bs3):
    grid_spec = pltpu.PrefetchScalarGridSpec(
        num_scalar_prefetch=2,
        grid=(HB,),
        in_specs=[pl.BlockSpec(memory_space=pltpu.HBM)],
        out_specs=pl.BlockSpec(memory_space=pltpu.HBM),
        scratch_shapes=[
            pltpu.VMEM((3, TCH, B, B * KS), jnp.float32),
            pltpu.VMEM((3, B, W * KS), jnp.float32),
            pltpu.SemaphoreType.DMA((3,)),
            pltpu.SemaphoreType.DMA((3,)),
        ],
    )
    return pl.pallas_call(
        _tc_body,
        grid_spec=grid_spec,
        out_shape=jax.ShapeDtypeStruct((H, W * KS), jnp.float32),
        compiler_params=pltpu.CompilerParams(
            dimension_semantics=("arbitrary",)),
    )(off, cs, bs3)


def kernel(block_indices, block_values):
    rows = block_indices[:, 0]
    cols = block_indices[:, 1]
    bv2 = block_values.reshape(N, BLK)
    off, cs, pos = _sc_sort()(rows, cols)
    bs = _sc_permute()(pos, bv2)
    out2 = _tc_accum(off, cs, bs.reshape(PAD_N, B, B * KS))
    return out2.reshape(H, W, KS)


# speedup vs baseline: 1.1762x; 1.0780x over previous
import functools

import jax
import jax.numpy as jnp
from jax import lax
from jax.experimental import pallas as pl
from jax.experimental.pallas import tpu as pltpu
from jax.experimental.pallas import tpu_sc as plsc

H = 4096
W = 4096
KS = 4
B = 16
HB = H // B
WB = W // B
N = 8192
BLK = B * B * KS
PAD_N = N + 64
NB = 2 * HB
OFF_LEN = 528

NT = 16
CHUNK1 = N // NT

NW = 32
RPW = N // NW
CH2 = 64
PH = PAD_N // 2
TCR = 16


def _ranks(rows_v, tmp_v, lanes, i):
    rv = rows_v[pl.ds(i * 16, 16)]
    srt, lids = plsc.sort_key_val(rv, lanes)
    tmp_v[...] = srt
    prev = plsc.load_gather(tmp_v, [jnp.maximum(lanes - 1, 0)])
    nxt = plsc.load_gather(tmp_v, [jnp.minimum(lanes + 1, 15)])
    segs = (lanes == 0) | (srt != prev)
    last = (lanes == 15) | (srt != nxt)
    rank = lanes - plsc.cummax(jnp.where(segs, lanes, 0))
    return srt, lids, rank, last


def _sc_sort_body(rows_hbm, cols_hbm, off_hbm, cs_hbm, pos_hbm,
                  rows_v, cols_v, hist_v, base_v, pos_v, off_v,
                  allh_v, tmp_v, hists_sh, sem):
    t = lax.axis_index("s")
    base0 = t * CHUNK1
    pltpu.sync_copy(rows_hbm.at[pl.ds(base0, CHUNK1)], rows_v)
    pltpu.sync_copy(cols_hbm.at[pl.ds(base0, CHUNK1)], cols_v)

    lanes = lax.iota(jnp.int32, 16)
    zero16 = jnp.zeros((16,), jnp.int32)
    for k in range(CHUNK1 // 16):
        cv = cols_v[pl.ds(k * 16, 16)]
        rows_v[pl.ds(k * 16, 16)] = (
            rows_v[pl.ds(k * 16, 16)] * 2 + (cv & 1))
        cols_v[pl.ds(k * 16, 16)] = (cv >> 1) * 128
    for k in range(NB // 16):
        hist_v[pl.ds(k * 16, 16)] = zero16

    def hist_body(i, c):
        srt, _, rank, last = _ranks(rows_v, tmp_v, lanes, i)
        plsc.addupdate_scatter(hist_v, [srt], rank + 1, mask=last)
        return c
    lax.fori_loop(0, CHUNK1 // 16, hist_body, 0)

    pltpu.sync_copy(hist_v, hists_sh.at[t])
    plsc.subcore_barrier()
    pltpu.sync_copy(hists_sh, allh_v)

    run = jnp.int32(0)
    for rg in range(NB // 16):
        tot = zero16
        before = zero16
        for tp in range(NT):
            row = allh_v[tp, pl.ds(rg * 16, 16)]
            tot = tot + row
            before = before + row * (jnp.int32(tp) < t).astype(jnp.int32)
        excl = plsc.cumsum(tot) - tot
        off_v[pl.ds(rg * 16, 16)] = run + excl
        base_v[pl.ds(rg * 16, 16)] = run + excl + before
        run = run + jnp.sum(tot)
    for k in range(NB, OFF_LEN, 16):
        off_v[pl.ds(k, 16)] = jnp.full((16,), N, jnp.int32)

    def pos_body(i, c):
        srt, lids, rank, last = _ranks(rows_v, tmp_v, lanes, i)
        p = plsc.load_gather(base_v, [srt]) + rank
        plsc.addupdate_scatter(base_v, [srt], rank + 1, mask=last)
        plsc.store_scatter(pos_v, [i * 16 + lids], p)
        return c
    lax.fori_loop(0, CHUNK1 // 16, pos_body, 0)

    @pl.when(t == 0)
    def _():
        pltpu.sync_copy(off_v, off_hbm)

    pltpu.async_copy(cols_v, cs_hbm.at[pos_v], sem).wait()
    pltpu.sync_copy(pos_v, pos_hbm.at[pl.ds(base0, CHUNK1)])


@functools.cache
def _sc_sort():
  return pl.kernel(
    _sc_sort_body,
    out_type=(jax.ShapeDtypeStruct((OFF_LEN,), jnp.int32),
              jax.ShapeDtypeStruct((N,), jnp.int32),
              jax.ShapeDtypeStruct((N,), jnp.int32)),
    mesh=plsc.VectorSubcoreMesh(core_axis_name="c", subcore_axis_name="s",
                                num_cores=1, num_subcores=NT),
    scratch_types=[
        pltpu.VMEM((CHUNK1,), jnp.int32),
        pltpu.VMEM((CHUNK1,), jnp.int32),
        pltpu.VMEM((NB,), jnp.int32),
        pltpu.VMEM((NB,), jnp.int32),
        pltpu.VMEM((CHUNK1,), jnp.int32),
        pltpu.VMEM((OFF_LEN,), jnp.int32),
        pltpu.VMEM((NT, NB), jnp.int32),
        pltpu.VMEM((16,), jnp.int32),
        pltpu.VMEM_SHARED((NT, NB), jnp.int32),
        pltpu.SemaphoreType.DMA,
    ],
    compiler_params=pltpu.CompilerParams(needs_layout_passes=False),
  )


def _sc_permute_body(pos_hbm, bv_hbm, bs_hbm, posc_v, buf_v, sem):
    c = lax.axis_index("c")
    s = lax.axis_index("s")
    wid = s * 2 + c
    for k in range(RPW // CH2):
        base = wid * RPW + k * CH2
        pltpu.sync_copy(pos_hbm.at[pl.ds(base, CH2)], posc_v)
        pltpu.sync_copy(bv_hbm.at[pl.ds(base, CH2)], buf_v)
        pltpu.async_copy(buf_v, bs_hbm.at[posc_v], sem).wait()


@functools.cache
def _sc_permute():
  return pl.kernel(
    _sc_permute_body,
    out_type=jax.ShapeDtypeStruct((PAD_N, BLK), jnp.float32),
    mesh=plsc.VectorSubcoreMesh(core_axis_name="c", subcore_axis_name="s",
                                num_cores=2, num_subcores=NT),
    scratch_types=[
        pltpu.VMEM((CH2,), jnp.int32),
        pltpu.VMEM((CH2, BLK), jnp.float32),
        pltpu.SemaphoreType.DMA,
    ],
  )


def _tc_body(off_smem, cs_smem, bs_any, out_any, chunk_v, acc_v,
             sem_in, sem_out):
    i = pl.program_id(0)
    n0 = off_smem[2 * i]
    nmid = off_smem[2 * i + 1]
    n1 = off_smem[2 * i + 2]
    r0 = n0 // 2
    nrows = (n1 + 1) // 2 - r0
    nch = (nrows + TCR - 1) // TCR
    sel = i % 3

    @pl.when((i == 0) & (nch > 0))
    def _():
        pltpu.make_async_copy(bs_any.at[pl.ds(r0, TCR)], chunk_v.at[2],
                              sem_in.at[2]).start()

    @pl.when(i >= 3)
    def _():
        pltpu.make_async_copy(acc_v.at[sel],
                              out_any.at[pl.ds((i - 3) * B, B)],
                              sem_out.at[sel]).wait()

    acc_v[sel] = jnp.zeros((B, W * KS), jnp.float32)
    zhalf = jnp.zeros((B, B * KS), jnp.float32)

    def chunk_body(ci, carry):
        csel = jnp.where(ci == 0, 2, (ci - 1) % 2)
        rbase = r0 + ci * TCR
        pltpu.make_async_copy(bs_any.at[pl.ds(rbase, TCR)], chunk_v.at[csel],
                              sem_in.at[csel]).wait()

        @pl.when(ci + 1 < nch)
        def _():
            pltpu.make_async_copy(bs_any.at[pl.ds(rbase + TCR, TCR)],
                                  chunk_v.at[ci % 2],
                                  sem_in.at[ci % 2]).start()

        mrows = jnp.minimum(nrows - ci * TCR, TCR)

        def q_body(q2, carry2):
            for kr in range(2):
                rowq = q2 * 2 + kr
                pairv = chunk_v[csel, jnp.minimum(rowq, TCR - 1)]
                halves = (pairv[:, :B * KS], pairv[:, B * KS:])
                for h in range(2):
                    g = 2 * (rbase + rowq) + h
                    valid = (g >= n0) & (g < n1) & (rowq < mrows)
                    lo = pl.multiple_of(cs_smem[jnp.minimum(g, N - 1)], 128)
                    blk = halves[h]
                    blk128 = jnp.where(
                        g < nmid,
                        jnp.concatenate([blk, zhalf], axis=1),
                        jnp.concatenate([zhalf, blk], axis=1))
                    blk128 = jnp.where(valid, blk128, 0.0)
                    acc_v[sel, :, pl.ds(lo, 128)] = (
                        acc_v[sel, :, pl.ds(lo, 128)] + blk128)
            return carry2
        lax.fori_loop(0, (mrows + 1) // 2, q_body, 0)
        return carry
    lax.fori_loop(0, nch, chunk_body, 0)

    @pl.when(i + 1 < HB)
    def _():
        n0n = off_smem[2 * i + 2]

        @pl.when(off_smem[2 * i + 4] > n0n)
        def _():
            pltpu.make_async_copy(bs_any.at[pl.ds(n0n // 2, TCR)],
                                  chunk_v.at[2], sem_in.at[2]).start()

    pltpu.make_async_copy(acc_v.at[sel], out_any.at[pl.ds(i * B, B)],
                          sem_out.at[sel]).start()

    @pl.when(i == HB - 1)
    def _():
        for d in range(3):
            pltpu.make_async_copy(acc_v.at[(i - d) % 3],
                                  out_any.at[pl.ds((i - d) * B, B)],
                                  sem_out.at[(i - d) % 3]).wait()


def _tc_accum(off, cs, bs3):
    grid_spec = pltpu.PrefetchScalarGridSpec(
        num_scalar_prefetch=2,
        grid=(HB,),
        in_specs=[pl.BlockSpec(memory_space=pltpu.HBM)],
        out_specs=pl.BlockSpec(memory_space=pltpu.HBM),
        scratch_shapes=[
            pltpu.VMEM((3, TCR, B, 2 * B * KS), jnp.float32),
            pltpu.VMEM((3, B, W * KS), jnp.float32),
            pltpu.SemaphoreType.DMA((3,)),
            pltpu.SemaphoreType.DMA((3,)),
        ],
    )
    return pl.pallas_call(
        _tc_body,
        grid_spec=grid_spec,
        out_shape=jax.ShapeDtypeStruct((H, W * KS), jnp.float32),
        compiler_params=pltpu.CompilerParams(
            dimension_semantics=("arbitrary",)),
    )(off, cs, bs3)


def kernel(block_indices, block_values):
    rows = block_indices[:, 0]
    cols = block_indices[:, 1]
    bv2 = block_values.reshape(N, BLK)
    off, cs, pos = _sc_sort()(rows, cols)
    bs = _sc_permute()(pos, bv2)
    out2 = _tc_accum(off, cs, bs.reshape(PH, B, 2 * B * KS))
    return out2.reshape(H, W, KS)
